# baseline (speedup 1.0000x reference)
import jax
import jax.numpy as jnp
from jax import lax
from jax.experimental import pallas as pl
from jax.experimental.pallas import tpu as pltpu
from jax.experimental.pallas import tpu_sc as plsc

N0 = 128
F1 = 8
F2 = 8
D = 6424
H = 256

KTC = 64
KSC = N0 - KTC
SOFF = KTC * F1

NC = 2
NS = 16
NW = NC * NS
NSEG = KSC * F1
SPW = NSEG // NW
L = 16
DP = ((D + L - 1) // L) * L
NVE = DP // L


def _sc_mean_body(x2_hbm, out_hbm, inbuf, outbuf, sem):
    wid = lax.axis_index("s") * NC + lax.axis_index("c")
    base = wid * SPW

    def fetch(i, slot):
        row0 = pl.multiple_of((SOFF + base + i) * F2, 8)
        return pltpu.async_copy(
            x2_hbm.at[pl.ds(row0, F2), :], inbuf.at[slot], sem)

    cps = {0: fetch(0, 0)}
    out_cps = {}
    for i in range(SPW):
        slot = i % 2
        if i + 1 < SPW:
            cps[i + 1] = fetch(i + 1, 1 - slot)
        cps.pop(i).wait()
        if i - 2 in out_cps:
            out_cps.pop(i - 2).wait()

        def slice_body(j, c2, slot=slot):
            for u in range(2):
                off = (2 * j + u) * L
                ld = [inbuf[slot, r, pl.ds(off, L)] for r in range(F2)]
                t0 = ld[0] + ld[1]
                t1 = ld[2] + ld[3]
                t2 = ld[4] + ld[5]
                t3 = ld[6] + ld[7]
                outbuf[pl.ds(slot * DP + off, L)] = (
                    (t0 + t1) + (t2 + t3)) * (1.0 / F2)
            return c2

        lax.fori_loop(0, NVE // 2, slice_body, 0)
        dst_off = pl.multiple_of((base + i) * D, 8)
        out_cps[i] = pltpu.async_copy(
            outbuf.at[pl.ds(slot * DP, D)], out_hbm.at[pl.ds(dst_off, D)], sem)
    for cp in out_cps.values():
        cp.wait()


def _sc_mean_x2(x2):
    mesh = plsc.VectorSubcoreMesh(core_axis_name="c", subcore_axis_name="s")
    flat = pl.kernel(
        _sc_mean_body,
        mesh=mesh,
        out_type=jax.ShapeDtypeStruct((NSEG * D,), jnp.float32),
        scratch_types=[
            pltpu.VMEM((2, F2, D), jnp.float32),
            pltpu.VMEM((2 * DP,), jnp.float32),
            pltpu.SemaphoreType.DMA,
        ],
        compiler_params=pltpu.CompilerParams(use_tc_tiling_on_sc=True),
    )(x2)
    return flat.reshape(NSEG, D)


B = 8
E = B * F1


def _epilogue(big, h0, ws1, wa1, w1top, b1, lng, lnb, w2, b2, out_ref, h1):
    f32 = jnp.float32
    mh1 = h1.reshape(B, F1, H).sum(axis=1) * (1.0 / F1)
    g0 = (jnp.dot(h0, ws1, preferred_element_type=f32)
          + jnp.dot(mh1, wa1, preferred_element_type=f32))
    t = jnp.dot(g0, w1top, preferred_element_type=f32)
    trep = jnp.broadcast_to(t[:, None, :], (B, F1, H)).reshape(E, H)

    e = big[:, H:] + trep + b1
    mu = e.mean(axis=-1, keepdims=True)
    var = ((e - mu) ** 2).mean(axis=-1, keepdims=True)
    hn = (e - mu) * jax.lax.rsqrt(var + 1e-5) * lng + lnb
    hn = jnp.maximum(hn, 0.0)
    out_ref[...] = (jnp.dot(hn, w2, preferred_element_type=f32) + b2)


def _stage1(x0_ref, x1_ref, wbig_ref, wa0_ref, m2):
    f32 = jnp.float32
    bf16 = jnp.bfloat16
    x1b = x1_ref[...]
    xs1 = x1b.reshape(E, D)
    m1 = x1b.sum(axis=1) * (1.0 / F1)

    wbig = wbig_ref[...]
    ws0 = wbig[:, :H]
    wa0 = wa0_ref[...]

    h0 = jnp.maximum(
        jnp.dot(x0_ref[...].astype(bf16), ws0, preferred_element_type=f32)
        + jnp.dot(m1.astype(bf16), wa0, preferred_element_type=f32), 0.0)
    big = jnp.dot(xs1.astype(bf16), wbig, preferred_element_type=f32)
    h1 = jnp.maximum(
        big[:, :H] + jnp.dot(m2.astype(bf16), wa0,
                             preferred_element_type=f32), 0.0)
    return big, h0, h1


def _tc_full_body(x0_ref, x1_ref, x2_ref, wbig_ref, wa0_ref,
                  ws1_ref, wa1_ref, w1top_ref, b1_ref, lng_ref, lnb_ref,
                  w2_ref, b2_ref, out_ref):
    m2 = x2_ref[:, 0, :]
    for j in range(1, F2):
        m2 = m2 + x2_ref[:, j, :]
    m2 = m2 * (1.0 / F2)
    big, h0, h1 = _stage1(x0_ref, x1_ref, wbig_ref, wa0_ref, m2)
    _epilogue(big, h0, ws1_ref[...], wa1_ref[...], w1top_ref[...],
              b1_ref[...], lng_ref[...], lnb_ref[...], w2_ref[...],
              b2_ref[...], out_ref, h1)


def _tc_tail_body(x0_ref, x1_ref, m2_ref, wbig_ref, wa0_ref,
                  ws1_ref, wa1_ref, w1top_ref, b1_ref, lng_ref, lnb_ref,
                  w2_ref, b2_ref, out_ref):
    big, h0, h1 = _stage1(x0_ref, x1_ref, wbig_ref, wa0_ref, m2_ref[...])
    _epilogue(big, h0, ws1_ref[...], wa1_ref[...], w1top_ref[...],
              b1_ref[...], lng_ref[...], lnb_ref[...], w2_ref[...],
              b2_ref[...], out_ref, h1)


def kernel(x0, x1, x2, W_self0, W_agg0, W_self1, W_agg1,
           mlp_w1, mlp_b1, ln_g, ln_b, mlp_w2, mlp_b2):
    m2 = _sc_mean_x2(x2)

    x1v = x1.reshape(N0, F1, D)
    x2v = x2.reshape(N0 * F1, F2, D)
    bf16 = jnp.bfloat16
    wbig = jnp.concatenate([W_self0, mlp_w1[H:]], axis=1).astype(bf16)
    wa0 = W_agg0.astype(bf16)
    w1top = mlp_w1[:H]
    b1 = mlp_b1.reshape(1, H)
    lng = ln_g.reshape(1, H)
    lnb = ln_b.reshape(1, H)
    b2 = mlp_b2.reshape(1, 1)

    full = lambda shape: pl.BlockSpec(shape, lambda i: (0,) * len(shape))
    wspecs = [
        full((D, 2 * H)),
        full((D, H)),
        full((H, H)),
        full((H, H)),
        full((H, H)),
        full((1, H)),
        full((1, H)),
        full((1, H)),
        full((H, 1)),
        full((1, 1)),
    ]
    wargs = (wbig, wa0, W_self1, W_agg1, w1top, b1, lng, lnb, mlp_w2, b2)

    out1 = pl.pallas_call(
        _tc_full_body,
        grid=(KTC // B,),
        in_specs=[
            pl.BlockSpec((B, D), lambda i: (i, 0)),
            pl.BlockSpec((B, F1, D), lambda i: (i, 0, 0)),
            pl.BlockSpec((E, F2, D), lambda i: (i, 0, 0)),
        ] + wspecs,
        out_specs=pl.BlockSpec((E, 1), lambda i: (i, 0)),
        out_shape=jax.ShapeDtypeStruct((KTC * F1, 1), jnp.float32),
        compiler_params=pltpu.CompilerParams(
            dimension_semantics=("arbitrary",),
        ),
    )(x0, x1v, x2v, *wargs)

    nb = KTC // B
    out2 = pl.pallas_call(
        _tc_tail_body,
        grid=(KSC // B,),
        in_specs=[
            pl.BlockSpec((B, D), lambda i, nb=nb: (i + nb, 0)),
            pl.BlockSpec((B, F1, D), lambda i, nb=nb: (i + nb, 0, 0)),
            pl.BlockSpec((E, D), lambda i: (i, 0)),
        ] + wspecs,
        out_specs=pl.BlockSpec((E, 1), lambda i: (i, 0)),
        out_shape=jax.ShapeDtypeStruct((KSC * F1, 1), jnp.float32),
        compiler_params=pltpu.CompilerParams(
            dimension_semantics=("arbitrary",),
        ),
    )(x0, x1v, m2, *wargs)
    return jnp.concatenate([out1, out2], axis=0)
